# baseline (speedup 1.0000x reference)
import functools

import jax
import jax.numpy as jnp
from jax import lax
from jax.experimental import pallas as pl
from jax.experimental.pallas import tpu as pltpu
from jax.experimental.pallas import tpu_sc as plsc

_NUM_EXPERTS = 16
_TOP_K = 2
_NUM_TOKENS = 16384
_N = _NUM_TOKENS * _TOP_K
_TILES = 16
_PER_TILE = _N // _TILES
_LANES = 16
_STEPS = _PER_TILE // _LANES


def _body(ids_hbm, scores_hbm, out_scores, out_idx, out_counts,
          ids_v, scores_v, lin_v, pos_v, ctab, alltot_v, tot_v, counts_v,
          tiletot_sh, sem):
    wid = lax.axis_index("s")
    base = wid * _PER_TILE
    iota = lax.iota(jnp.int32, _LANES)
    zeros16 = jnp.zeros((_LANES,), jnp.int32)
    ones16 = jnp.ones((_LANES,), jnp.int32)
    lane_row = iota * _NUM_EXPERTS

    pltpu.sync_copy(ids_hbm.at[pl.ds(base, _PER_TILE)], ids_v)
    pltpu.sync_copy(scores_hbm.at[pl.ds(base, _PER_TILE)], scores_v)

    for k in range(_LANES):
        ctab[pl.ds(k * _LANES, _LANES)] = zeros16

    def hist_body(t, carry):
        lidx = iota * _STEPS + t
        v = plsc.load_gather(ids_v, [lidx])
        plsc.addupdate_scatter(ctab, [lane_row + v], ones16)
        lin_v[pl.ds(t * _LANES, _LANES)] = base + t * _LANES + iota
        return carry

    lax.fori_loop(0, _STEPS, hist_body, 0)

    tot = zeros16
    for k in range(_LANES):
        tot = tot + ctab[pl.ds(k * _LANES, _LANES)]
    tot_v[...] = tot
    pltpu.sync_copy(tot_v, tiletot_sh.at[pl.ds(wid * _LANES, _LANES)])
    plsc.subcore_barrier()
    pltpu.sync_copy(tiletot_sh, alltot_v)

    wid_vec = zeros16 + wid
    acc = zeros16
    my_base = zeros16
    for w in range(_TILES):
        my_base = jnp.where(wid_vec == w, acc, my_base)
        acc = acc + alltot_v[pl.ds(w * _LANES, _LANES)]
    total = acc
    expert_off = plsc.cumsum(total) - total
    cb = expert_off + my_base
    for j in range(_LANES):
        row = ctab[pl.ds(j * _LANES, _LANES)]
        ctab[pl.ds(j * _LANES, _LANES)] = cb
        cb = cb + row

    def rank_body(t, carry):
        lidx = iota * _STEPS + t
        v = plsc.load_gather(ids_v, [lidx])
        cidx = lane_row + v
        p = plsc.load_gather(ctab, [cidx])
        plsc.addupdate_scatter(ctab, [cidx], ones16)
        plsc.store_scatter(pos_v, [iota, zeros16 + t], p)
        return carry

    lax.fori_loop(0, _STEPS, rank_body, 0)

    copies = []
    for j in range(_LANES):
        copies.append(pltpu.make_async_copy(
            scores_v.at[pl.ds(j * _STEPS, _STEPS)],
            out_scores.at[pos_v.at[j]], sem))
        copies.append(pltpu.make_async_copy(
            lin_v.at[pl.ds(j * _STEPS, _STEPS)],
            out_idx.at[pos_v.at[j]], sem))
    for c in copies:
        c.start()
    for c in copies:
        c.wait()

    @pl.when(wid == 0)
    def _():
        counts_v[...] = total.astype(jnp.float32)
        pltpu.sync_copy(counts_v, out_counts)


@functools.partial(
    pl.kernel,
    out_type=[
        jax.ShapeDtypeStruct((_N,), jnp.float32),
        jax.ShapeDtypeStruct((_N,), jnp.int32),
        jax.ShapeDtypeStruct((_NUM_EXPERTS,), jnp.float32),
    ],
    mesh=plsc.VectorSubcoreMesh(
        core_axis_name="c", subcore_axis_name="s", num_cores=1),
    compiler_params=pltpu.CompilerParams(needs_layout_passes=False),
    scratch_types=[
        pltpu.VMEM((_PER_TILE,), jnp.int32),
        pltpu.VMEM((_PER_TILE,), jnp.float32),
        pltpu.VMEM((_PER_TILE,), jnp.int32),
        pltpu.VMEM((_LANES, _STEPS), jnp.int32),
        pltpu.VMEM((_NUM_EXPERTS * _LANES,), jnp.int32),
        pltpu.VMEM((_TILES * _LANES,), jnp.int32),
        pltpu.VMEM((_LANES,), jnp.int32),
        pltpu.VMEM((_NUM_EXPERTS,), jnp.float32),
        pltpu.VMEM_SHARED((_TILES * _LANES,), jnp.int32),
        pltpu.SemaphoreType.DMA,
    ],
)
def _reorder(ids_hbm, scores_hbm, out_scores, out_idx, out_counts, *scratch):
    _body(ids_hbm, scores_hbm, out_scores, out_idx, out_counts, *scratch)


def kernel(top_scores, selected_experts_indices):
    ids = selected_experts_indices.reshape(-1)
    scores = top_scores.reshape(-1)
    return tuple(_reorder(ids, scores))


# speedup vs baseline: 2.8580x; 2.8580x over previous
import functools

import jax
import jax.numpy as jnp
from jax import lax
from jax.experimental import pallas as pl
from jax.experimental.pallas import tpu as pltpu
from jax.experimental.pallas import tpu_sc as plsc

_NUM_EXPERTS = 16
_TOP_K = 2
_NUM_TOKENS = 16384
_N = _NUM_TOKENS * _TOP_K
_TILES = 16
_PER_TILE = _N // _TILES
_LANES = 16
_STEPS = _PER_TILE // _LANES


def _body(ids_hbm, scores_hbm, out_scores, out_idx, out_counts,
          ids_v, scores_v, lin_v, pos_v, ctab, alltot_v, tot_v, counts_v,
          tiletot_sh, scores_sh, idx_sh, sem):
    wid = lax.axis_index("s")
    base = wid * _PER_TILE
    iota = lax.iota(jnp.int32, _LANES)
    zeros16 = jnp.zeros((_LANES,), jnp.int32)
    ones16 = jnp.ones((_LANES,), jnp.int32)
    lane_row = iota * _NUM_EXPERTS

    pltpu.sync_copy(ids_hbm.at[pl.ds(base, _PER_TILE)], ids_v)
    pltpu.sync_copy(scores_hbm.at[pl.ds(base, _PER_TILE)], scores_v)

    for k in range(_LANES):
        ctab[pl.ds(k * _LANES, _LANES)] = zeros16

    def hist_body(t, carry):
        lidx = iota * _STEPS + t
        v = plsc.load_gather(ids_v, [lidx])
        plsc.addupdate_scatter(ctab, [lane_row + v], ones16)
        lin_v[pl.ds(t * _LANES, _LANES)] = base + t * _LANES + iota
        return carry

    lax.fori_loop(0, _STEPS, hist_body, 0)

    tot = zeros16
    for k in range(_LANES):
        tot = tot + ctab[pl.ds(k * _LANES, _LANES)]
    tot_v[...] = tot
    pltpu.sync_copy(tot_v, tiletot_sh.at[pl.ds(wid * _LANES, _LANES)])
    plsc.subcore_barrier()
    pltpu.sync_copy(tiletot_sh, alltot_v)

    wid_vec = zeros16 + wid
    acc = zeros16
    my_base = zeros16
    for w in range(_TILES):
        my_base = jnp.where(wid_vec == w, acc, my_base)
        acc = acc + alltot_v[pl.ds(w * _LANES, _LANES)]
    total = acc
    expert_off = plsc.cumsum(total) - total
    cb = expert_off + my_base
    for j in range(_LANES):
        row = ctab[pl.ds(j * _LANES, _LANES)]
        ctab[pl.ds(j * _LANES, _LANES)] = cb
        cb = cb + row

    def rank_body(t, carry):
        lidx = iota * _STEPS + t
        v = plsc.load_gather(ids_v, [lidx])
        cidx = lane_row + v
        p = plsc.load_gather(ctab, [cidx])
        plsc.addupdate_scatter(ctab, [cidx], ones16)
        plsc.store_scatter(pos_v, [iota, zeros16 + t], p)
        return carry

    lax.fori_loop(0, _STEPS, rank_body, 0)

    copies = []
    for j in range(_LANES):
        copies.append(pltpu.make_async_copy(
            scores_v.at[pl.ds(j * _STEPS, _STEPS)],
            scores_sh.at[pos_v.at[j]], sem))
        copies.append(pltpu.make_async_copy(
            lin_v.at[pl.ds(j * _STEPS, _STEPS)],
            idx_sh.at[pos_v.at[j]], sem))
    for c in copies:
        c.start()
    for c in copies:
        c.wait()
    plsc.subcore_barrier()
    pltpu.sync_copy(scores_sh.at[pl.ds(base, _PER_TILE)],
                    out_scores.at[pl.ds(base, _PER_TILE)])
    pltpu.sync_copy(idx_sh.at[pl.ds(base, _PER_TILE)],
                    out_idx.at[pl.ds(base, _PER_TILE)])

    @pl.when(wid == 0)
    def _():
        counts_v[...] = total.astype(jnp.float32)
        pltpu.sync_copy(counts_v, out_counts)


@functools.partial(
    pl.kernel,
    out_type=[
        jax.ShapeDtypeStruct((_N,), jnp.float32),
        jax.ShapeDtypeStruct((_N,), jnp.int32),
        jax.ShapeDtypeStruct((_NUM_EXPERTS,), jnp.float32),
    ],
    mesh=plsc.VectorSubcoreMesh(
        core_axis_name="c", subcore_axis_name="s", num_cores=1),
    compiler_params=pltpu.CompilerParams(needs_layout_passes=False),
    scratch_types=[
        pltpu.VMEM((_PER_TILE,), jnp.int32),
        pltpu.VMEM((_PER_TILE,), jnp.float32),
        pltpu.VMEM((_PER_TILE,), jnp.int32),
        pltpu.VMEM((_LANES, _STEPS), jnp.int32),
        pltpu.VMEM((_NUM_EXPERTS * _LANES,), jnp.int32),
        pltpu.VMEM((_TILES * _LANES,), jnp.int32),
        pltpu.VMEM((_LANES,), jnp.int32),
        pltpu.VMEM((_NUM_EXPERTS,), jnp.float32),
        pltpu.VMEM_SHARED((_TILES * _LANES,), jnp.int32),
        pltpu.VMEM_SHARED((_N,), jnp.float32),
        pltpu.VMEM_SHARED((_N,), jnp.int32),
        pltpu.SemaphoreType.DMA,
    ],
)
def _reorder(ids_hbm, scores_hbm, out_scores, out_idx, out_counts, *scratch):
    _body(ids_hbm, scores_hbm, out_scores, out_idx, out_counts, *scratch)


def kernel(top_scores, selected_experts_indices):
    ids = selected_experts_indices.reshape(-1)
    scores = top_scores.reshape(-1)
    return tuple(_reorder(ids, scores))


# speedup vs baseline: 3.1792x; 1.1124x over previous
import functools

import jax
import jax.numpy as jnp
from jax import lax
from jax.experimental import pallas as pl
from jax.experimental.pallas import tpu as pltpu
from jax.experimental.pallas import tpu_sc as plsc

_NUM_EXPERTS = 16
_TOP_K = 2
_NUM_TOKENS = 16384
_N = _NUM_TOKENS * _TOP_K
_TILES = 16
_PER_TILE = _N // _TILES
_LANES = 16
_STEPS = _PER_TILE // _LANES

_MESH = plsc.VectorSubcoreMesh(
    core_axis_name="c", subcore_axis_name="s", num_cores=1)
_CPARAMS = pltpu.CompilerParams(needs_layout_passes=False)


def _sort_body(ids_hbm, out_idx, out_counts,
               ids_v, lin_v, pos_v, ctab, alltot_v, tot_v, counts_v,
               tiletot_sh, idx_sh, sem):
    wid = lax.axis_index("s")
    base = wid * _PER_TILE
    iota = lax.iota(jnp.int32, _LANES)
    zeros16 = jnp.zeros((_LANES,), jnp.int32)
    ones16 = jnp.ones((_LANES,), jnp.int32)
    lane_row = iota * _NUM_EXPERTS

    pltpu.sync_copy(ids_hbm.at[pl.ds(base, _PER_TILE)], ids_v)

    for k in range(_LANES):
        ctab[pl.ds(k * _LANES, _LANES)] = zeros16

    def hist_body(t, carry):
        lidx = iota * _STEPS + t
        v = plsc.load_gather(ids_v, [lidx])
        plsc.addupdate_scatter(ctab, [lane_row + v], ones16)
        lin_v[pl.ds(t * _LANES, _LANES)] = base + t * _LANES + iota
        return carry

    lax.fori_loop(0, _STEPS, hist_body, 0)

    tot = zeros16
    for k in range(_LANES):
        tot = tot + ctab[pl.ds(k * _LANES, _LANES)]
    tot_v[...] = tot
    pltpu.sync_copy(tot_v, tiletot_sh.at[pl.ds(wid * _LANES, _LANES)])
    plsc.subcore_barrier()
    pltpu.sync_copy(tiletot_sh, alltot_v)

    wid_vec = zeros16 + wid
    acc = zeros16
    my_base = zeros16
    for w in range(_TILES):
        my_base = jnp.where(wid_vec == w, acc, my_base)
        acc = acc + alltot_v[pl.ds(w * _LANES, _LANES)]
    total = acc
    expert_off = plsc.cumsum(total) - total
    cb = expert_off + my_base
    for j in range(_LANES):
        row = ctab[pl.ds(j * _LANES, _LANES)]
        ctab[pl.ds(j * _LANES, _LANES)] = cb
        cb = cb + row

    def rank_body(t, carry):
        lidx = iota * _STEPS + t
        v = plsc.load_gather(ids_v, [lidx])
        cidx = lane_row + v
        p = plsc.load_gather(ctab, [cidx])
        plsc.addupdate_scatter(ctab, [cidx], ones16)
        plsc.store_scatter(pos_v, [iota, zeros16 + t], p)
        return carry

    lax.fori_loop(0, _STEPS, rank_body, 0)

    copies = []
    for j in range(_LANES):
        copies.append(pltpu.make_async_copy(
            lin_v.at[pl.ds(j * _STEPS, _STEPS)],
            idx_sh.at[pos_v.at[j]], sem))
    for c in copies:
        c.start()
    for c in copies:
        c.wait()
    plsc.subcore_barrier()
    pltpu.sync_copy(idx_sh.at[pl.ds(base, _PER_TILE)],
                    out_idx.at[pl.ds(base, _PER_TILE)])

    @pl.when(wid == 0)
    def _():
        counts_v[...] = total.astype(jnp.float32)
        pltpu.sync_copy(counts_v, out_counts)


@functools.partial(
    pl.kernel,
    out_type=[
        jax.ShapeDtypeStruct((_N,), jnp.int32),
        jax.ShapeDtypeStruct((_NUM_EXPERTS,), jnp.float32),
    ],
    mesh=_MESH,
    compiler_params=_CPARAMS,
    scratch_types=[
        pltpu.VMEM((_PER_TILE,), jnp.int32),
        pltpu.VMEM((_PER_TILE,), jnp.int32),
        pltpu.VMEM((_LANES, _STEPS), jnp.int32),
        pltpu.VMEM((_NUM_EXPERTS * _LANES,), jnp.int32),
        pltpu.VMEM((_TILES * _LANES,), jnp.int32),
        pltpu.VMEM((_LANES,), jnp.int32),
        pltpu.VMEM((_NUM_EXPERTS,), jnp.float32),
        pltpu.VMEM_SHARED((_TILES * _LANES,), jnp.int32),
        pltpu.VMEM_SHARED((_N,), jnp.int32),
        pltpu.SemaphoreType.DMA,
    ],
)
def _sort_ids(ids_hbm, out_idx, out_counts, *scratch):
    _sort_body(ids_hbm, out_idx, out_counts, *scratch)


def _gather_body(scores_hbm, idx_hbm, out_scores, sv, idxv, outv, sem):
    wid = lax.axis_index("s")
    base = wid * _PER_TILE
    pltpu.sync_copy(scores_hbm, sv)
    pltpu.sync_copy(idx_hbm.at[pl.ds(base, _PER_TILE)], idxv)

    def gb(t, carry):
        f = idxv[pl.ds(t * _LANES, _LANES)]
        outv[pl.ds(t * _LANES, _LANES)] = plsc.load_gather(sv, [f])
        return carry

    lax.fori_loop(0, _STEPS, gb, 0)
    pltpu.sync_copy(outv, out_scores.at[pl.ds(base, _PER_TILE)])


@functools.partial(
    pl.kernel,
    out_type=[jax.ShapeDtypeStruct((_N,), jnp.float32)],
    mesh=_MESH,
    compiler_params=_CPARAMS,
    scratch_types=[
        pltpu.VMEM((_N,), jnp.float32),
        pltpu.VMEM((_PER_TILE,), jnp.int32),
        pltpu.VMEM((_PER_TILE,), jnp.float32),
        pltpu.SemaphoreType.DMA,
    ],
)
def _gather_scores(scores_hbm, idx_hbm, out_scores, *scratch):
    _gather_body(scores_hbm, idx_hbm, out_scores, *scratch)


def kernel(top_scores, selected_experts_indices):
    ids = selected_experts_indices.reshape(-1)
    scores = top_scores.reshape(-1)
    out_idx, out_counts = _sort_ids(ids)
    (out_scores,) = _gather_scores(scores, out_idx)
    return out_scores, out_idx, out_counts


# speedup vs baseline: 3.9755x; 1.2505x over previous
import functools

import jax
import jax.numpy as jnp
from jax import lax
from jax.experimental import pallas as pl
from jax.experimental.pallas import tpu as pltpu
from jax.experimental.pallas import tpu_sc as plsc

_NUM_EXPERTS = 16
_TOP_K = 2
_NUM_TOKENS = 16384
_N = _NUM_TOKENS * _TOP_K
_TILES = 16
_PER_TILE = _N // _TILES
_LANES = 16
_STEPS = _PER_TILE // _LANES

_MESH = plsc.VectorSubcoreMesh(
    core_axis_name="c", subcore_axis_name="s", num_cores=1)
_CPARAMS = pltpu.CompilerParams(needs_layout_passes=False)


def _sort_body(ids_hbm, out_idx, out_counts,
               ids_v, lin_v, pos_v, ctab, alltot_v, tot_v, counts_v,
               tiletot_sh, idx_sh, sem):
    wid = lax.axis_index("s")
    base = wid * _PER_TILE
    iota = lax.iota(jnp.int32, _LANES)
    zeros16 = jnp.zeros((_LANES,), jnp.int32)
    ones16 = jnp.ones((_LANES,), jnp.int32)
    lane_row = iota * _NUM_EXPERTS

    pltpu.sync_copy(ids_hbm.at[pl.ds(wid * 8, 8), :, :], ids_v)

    for k in range(_LANES):
        ctab[pl.ds(k * _LANES, _LANES)] = zeros16

    def hist_body(t, carry):
        lidx = iota * _STEPS + t
        il = lidx >> 1
        v = plsc.load_gather(ids_v, [il >> 7, lidx & 1, il & 127])
        plsc.addupdate_scatter(ctab, [lane_row + v], ones16)
        lin_v[pl.ds(t * _LANES, _LANES)] = base + t * _LANES + iota
        return carry

    lax.fori_loop(0, _STEPS, hist_body, 0)

    tot = zeros16
    for k in range(_LANES):
        tot = tot + ctab[pl.ds(k * _LANES, _LANES)]
    tot_v[...] = tot
    pltpu.sync_copy(tot_v, tiletot_sh.at[pl.ds(wid * _LANES, _LANES)])
    plsc.subcore_barrier()
    pltpu.sync_copy(tiletot_sh, alltot_v)

    wid_vec = zeros16 + wid
    acc = zeros16
    my_base = zeros16
    for w in range(_TILES):
        my_base = jnp.where(wid_vec == w, acc, my_base)
        acc = acc + alltot_v[pl.ds(w * _LANES, _LANES)]
    total = acc
    expert_off = plsc.cumsum(total) - total
    cb = expert_off + my_base
    for j in range(_LANES):
        row = ctab[pl.ds(j * _LANES, _LANES)]
        ctab[pl.ds(j * _LANES, _LANES)] = cb
        cb = cb + row

    def rank_body(t, carry):
        lidx = iota * _STEPS + t
        il = lidx >> 1
        v = plsc.load_gather(ids_v, [il >> 7, lidx & 1, il & 127])
        cidx = lane_row + v
        p = plsc.load_gather(ctab, [cidx])
        plsc.addupdate_scatter(ctab, [cidx], ones16)
        plsc.store_scatter(pos_v, [iota, zeros16 + t], p)
        return carry

    lax.fori_loop(0, _STEPS, rank_body, 0)

    copies = []
    for j in range(_LANES):
        copies.append(pltpu.make_async_copy(
            lin_v.at[pl.ds(j * _STEPS, _STEPS)],
            idx_sh.at[pos_v.at[j]], sem))
    for c in copies:
        c.start()
    for c in copies:
        c.wait()
    plsc.subcore_barrier()
    pltpu.sync_copy(idx_sh.at[pl.ds(base, _PER_TILE)],
                    out_idx.at[pl.ds(base, _PER_TILE)])

    @pl.when(wid == 0)
    def _():
        counts_v[...] = total.astype(jnp.float32)
        pltpu.sync_copy(counts_v, out_counts)


@functools.partial(
    pl.kernel,
    out_type=[
        jax.ShapeDtypeStruct((_N,), jnp.int32),
        jax.ShapeDtypeStruct((_NUM_EXPERTS,), jnp.float32),
    ],
    mesh=_MESH,
    compiler_params=_CPARAMS,
    scratch_types=[
        pltpu.VMEM((8, _TOP_K, 128), jnp.int32),
        pltpu.VMEM((_PER_TILE,), jnp.int32),
        pltpu.VMEM((_LANES, _STEPS), jnp.int32),
        pltpu.VMEM((_NUM_EXPERTS * _LANES,), jnp.int32),
        pltpu.VMEM((_TILES * _LANES,), jnp.int32),
        pltpu.VMEM((_LANES,), jnp.int32),
        pltpu.VMEM((_NUM_EXPERTS,), jnp.float32),
        pltpu.VMEM_SHARED((_TILES * _LANES,), jnp.int32),
        pltpu.VMEM_SHARED((_N,), jnp.int32),
        pltpu.SemaphoreType.DMA,
    ],
)
def _sort_ids(ids_hbm, out_idx, out_counts, *scratch):
    _sort_body(ids_hbm, out_idx, out_counts, *scratch)


def _gather_body(scores_hbm, idx_hbm, out_scores, sv, idxv, outv, sem):
    wid = lax.axis_index("s")
    base = wid * _PER_TILE
    pltpu.sync_copy(scores_hbm, sv)
    pltpu.sync_copy(idx_hbm.at[pl.ds(base, _PER_TILE)], idxv)

    def gb(t, carry):
        f = idxv[pl.ds(t * _LANES, _LANES)]
        outv[pl.ds(t * _LANES, _LANES)] = plsc.load_gather(sv, [f])
        return carry

    lax.fori_loop(0, _STEPS, gb, 0)
    pltpu.sync_copy(outv, out_scores.at[pl.ds(base, _PER_TILE)])


@functools.partial(
    pl.kernel,
    out_type=[jax.ShapeDtypeStruct((_N,), jnp.float32)],
    mesh=_MESH,
    compiler_params=_CPARAMS,
    scratch_types=[
        pltpu.VMEM((_N,), jnp.float32),
        pltpu.VMEM((_PER_TILE,), jnp.int32),
        pltpu.VMEM((_PER_TILE,), jnp.float32),
        pltpu.SemaphoreType.DMA,
    ],
)
def _gather_scores(scores_hbm, idx_hbm, out_scores, *scratch):
    _gather_body(scores_hbm, idx_hbm, out_scores, *scratch)


def kernel(top_scores, selected_experts_indices):
    ids3 = selected_experts_indices.reshape(128, 128, _TOP_K).transpose(0, 2, 1)
    scores = top_scores.reshape(-1)
    out_idx, out_counts = _sort_ids(ids3)
    (out_scores,) = _gather_scores(scores, out_idx)
    return out_scores, out_idx, out_counts
